# Initial kernel scaffold; baseline (speedup 1.0000x reference)
import jax
import jax.numpy as jnp
from jax.experimental import pallas as pl


def kernel(input_ids, table, ln_gamma, ln_beta):
    raise NotImplementedError("write your pallas kernel here")


# speedup vs baseline: 1.2133x; 1.2133x over previous
import functools

import numpy as np
import jax
import jax.numpy as jnp
from jax import lax
from jax.experimental import pallas as pl
from jax.experimental.pallas import tpu as pltpu
from jax.experimental.pallas import tpu_sc as plsc

_EPS = 1e-12


def _sinusoidal_pe(seq_len, dim):
    pos = np.arange(seq_len, dtype=np.float32)[:, None]
    i = np.arange(dim, dtype=np.float32)[None, :]
    angle = pos / np.power(10000.0, (2.0 * np.floor(i / 2.0)) / dim)
    pe = np.zeros((seq_len, dim), dtype=np.float32)
    pe[:, 0::2] = np.sin(angle[:, 0::2])
    pe[:, 1::2] = np.cos(angle[:, 1::2])
    return pe


_NW = 32


def _sc_gather(table, idx, chunk=64):
    n = idx.shape[0]
    d = table.shape[1]
    n_per_w = n // _NW
    n_chunks = n_per_w // chunk
    mesh = plsc.VectorSubcoreMesh(core_axis_name="c", subcore_axis_name="s")

    @functools.partial(
        pl.kernel,
        out_type=jax.ShapeDtypeStruct((n, d), table.dtype),
        mesh=mesh,
        scratch_types=[
            pltpu.VMEM((n_per_w,), jnp.int32),
            pltpu.VMEM((chunk, d), table.dtype),
        ],
    )
    def gather_kernel(table_hbm, i_hbm, o_hbm, idx_v, rows_v):
        wid = lax.axis_index("s") * 2 + lax.axis_index("c")
        base = wid * n_per_w
        pltpu.sync_copy(i_hbm.at[pl.ds(base, n_per_w)], idx_v)

        @pl.loop(0, n_chunks)
        def _(c):
            pltpu.sync_copy(table_hbm.at[idx_v.at[pl.ds(c * chunk, chunk)]], rows_v)
            pltpu.sync_copy(rows_v, o_hbm.at[pl.ds(base + c * chunk, chunk)])

    return gather_kernel(table, idx)


def _addln_body(x_ref, pe_ref, g_ref, b_ref, o_ref):
    x = x_ref[0] + pe_ref[...]
    mean = jnp.mean(x, axis=-1, keepdims=True)
    c = x - mean
    var = jnp.mean(c * c, axis=-1, keepdims=True)
    o_ref[0] = c * lax.rsqrt(var + _EPS) * g_ref[...] + b_ref[...]


def _tc_addln(gathered, pe, gamma, beta, bs=512):
    b, s, d = gathered.shape
    return pl.pallas_call(
        _addln_body,
        grid=(s // bs, b),
        in_specs=[
            pl.BlockSpec((1, bs, d), lambda i, j: (j, i, 0)),
            pl.BlockSpec((bs, d), lambda i, j: (i, 0)),
            pl.BlockSpec((1, d), lambda i, j: (0, 0)),
            pl.BlockSpec((1, d), lambda i, j: (0, 0)),
        ],
        out_specs=pl.BlockSpec((1, bs, d), lambda i, j: (j, i, 0)),
        out_shape=jax.ShapeDtypeStruct((b, s, d), jnp.float32),
    )(gathered, pe, gamma.reshape(1, d), beta.reshape(1, d))


def kernel(input_ids, table, ln_gamma, ln_beta):
    b, s = input_ids.shape
    d = table.shape[1]
    pe = jnp.asarray(_sinusoidal_pe(s, d))
    gathered = _sc_gather(table, input_ids.reshape(-1)).reshape(b, s, d)

    return _tc_addln(gathered, pe, ln_gamma, ln_beta)
